# Initial kernel scaffold; baseline (speedup 1.0000x reference)
import jax
import jax.numpy as jnp
from jax.experimental import pallas as pl


def kernel(x, y, src_emb, tgt_emb, W_ih_e0, W_hh_e0, b_e0, W_ih_e12, W_hh_e12, b_e12, W_ih_d0, W_hh_d0, b_d0, W_ih_d12, W_hh_d12, b_d12, W_a, W_out, b_out):
    raise NotImplementedError("write your pallas kernel here")


# speedup vs baseline: 3.1349x; 3.1349x over previous
import jax
import jax.numpy as jnp
from jax.experimental import pallas as pl
from jax.experimental.pallas import tpu as pltpu


def _dg(x, w):
    return jax.lax.dot_general(
        x, w, (((1,), (1,)), ((), ())), preferred_element_type=jnp.float32)


def _lstm_elem(g, c, fd):
    i = g[:, 0 * fd:1 * fd]
    f = g[:, 1 * fd:2 * fd]
    gg = g[:, 2 * fd:3 * fd]
    o = g[:, 3 * fd:4 * fd]
    c2 = jax.nn.sigmoid(f) * c + jax.nn.sigmoid(i) * jnp.tanh(gg)
    h2 = jax.nn.sigmoid(o) * jnp.tanh(c2)
    return h2, c2


def _bilstm_body(seq_ref, wih_ref, whh_ref, b_ref, out_ref, hf_ref, cf_ref):
    S, B, _ = seq_ref.shape
    Dh = whh_ref.shape[2]

    def cell(x, h, c, d):
        g = _dg(x, wih_ref[d]) + _dg(h, whh_ref[d]) + b_ref[d][None, :]
        return _lstm_elem(g, c, Dh)

    z = jnp.zeros((B, Dh), jnp.float32)

    def fwd_step(t, hc):
        h2, c2 = cell(seq_ref[t], hc[0], hc[1], 0)
        out_ref[t, :, :Dh] = h2
        return (h2, c2)

    hF, cF = jax.lax.fori_loop(0, S, fwd_step, (z, z))
    hf_ref[0] = hF
    cf_ref[0] = cF

    def bwd_step(t, hc):
        s = S - 1 - t
        h2, c2 = cell(seq_ref[s], hc[0], hc[1], 1)
        out_ref[s, :, Dh:] = h2
        return (h2, c2)

    hB, cB = jax.lax.fori_loop(0, S, bwd_step, (z, z))
    hf_ref[1] = hB
    cf_ref[1] = cB


def _bilstm_layer(seq, wih, whh, b):
    S, B, _ = seq.shape
    Dh = whh.shape[2]
    return pl.pallas_call(
        _bilstm_body,
        out_shape=(
            jax.ShapeDtypeStruct((S, B, 2 * Dh), jnp.float32),
            jax.ShapeDtypeStruct((2, B, Dh), jnp.float32),
            jax.ShapeDtypeStruct((2, B, Dh), jnp.float32),
        ),
        compiler_params=pltpu.CompilerParams(
            vmem_limit_bytes=50 * 1024 * 1024),
        name="bilstm_layer",
    )(seq, wih, whh, b)


def _dec_body(emb_ref, enc_ref, wa_ref, wih0_ref, whh0_ref, b0_ref,
              wih12_ref, whh12_ref, b12_ref, h_init_ref, c_init_ref, hs_ref):
    T, B, H = emb_ref.shape

    def step(t, carry):
        h0, c0, h1, c1, h2, c2 = carry
        emb = emb_ref[t]
        q = jnp.dot(h2, wa_ref[...], preferred_element_type=jnp.float32)
        enc = enc_ref[...]
        scores = jnp.sum(q[None, :, :] * enc, axis=2)
        m = jnp.max(scores, axis=0, keepdims=True)
        e = jnp.exp(scores - m)
        attn = e / jnp.sum(e, axis=0, keepdims=True)
        ctx = jnp.sum(attn[:, :, None] * enc, axis=0)

        inp = jnp.concatenate([emb, ctx], axis=1)
        g0 = _dg(inp, wih0_ref[...]) + _dg(h0, whh0_ref[...]) + b0_ref[...]
        h0n, c0n = _lstm_elem(g0, c0, H)
        g1 = (_dg(h0n, wih12_ref[0]) + _dg(h1, whh12_ref[0])
              + b12_ref[0][None, :])
        h1n, c1n = _lstm_elem(g1, c1, H)
        g2 = (_dg(h1n, wih12_ref[1]) + _dg(h2, whh12_ref[1])
              + b12_ref[1][None, :])
        h2n, c2n = _lstm_elem(g2, c2, H)
        hs_ref[t] = h2n
        return (h0n, c0n, h1n, c1n, h2n, c2n)

    init = (h_init_ref[0], c_init_ref[0], h_init_ref[1], c_init_ref[1],
            h_init_ref[2], c_init_ref[2])
    jax.lax.fori_loop(0, T, step, init)


def _decoder(emb_y, enc_seq, W_a, W_ih_d0, W_hh_d0, b0, W_ih_d12, W_hh_d12,
             b_d12, h_init, c_init):
    T, B, H = emb_y.shape
    return pl.pallas_call(
        _dec_body,
        out_shape=jax.ShapeDtypeStruct((T, B, H), jnp.float32),
        compiler_params=pltpu.CompilerParams(
            vmem_limit_bytes=55 * 1024 * 1024),
        name="decoder_recurrence",
    )(emb_y, enc_seq, W_a, W_ih_d0, W_hh_d0, b0, W_ih_d12, W_hh_d12,
      b_d12, h_init, c_init)


def _proj_body(x_ref, w_ref, b_ref, o_ref):
    o_ref[...] = _dg(x_ref[...], w_ref[...]) + b_ref[...]


def _projection(x, w, b):
    R, H = x.shape
    V = w.shape[0]
    BR, BV = R // 2, 3200
    return pl.pallas_call(
        _proj_body,
        out_shape=jax.ShapeDtypeStruct((R, V), jnp.float32),
        grid=(V // BV, R // BR),
        in_specs=[
            pl.BlockSpec((BR, H), lambda v, r: (r, 0)),
            pl.BlockSpec((BV, H), lambda v, r: (v, 0)),
            pl.BlockSpec((1, BV), lambda v, r: (0, v)),
        ],
        out_specs=pl.BlockSpec((BR, BV), lambda v, r: (r, v)),
        compiler_params=pltpu.CompilerParams(
            dimension_semantics=("parallel", "arbitrary"),
            vmem_limit_bytes=55 * 1024 * 1024),
        name="out_projection",
    )(x, w, b)


def kernel(x, y, src_emb, tgt_emb, W_ih_e0, W_hh_e0, b_e0, W_ih_e12,
           W_hh_e12, b_e12, W_ih_d0, W_hh_d0, b_d0, W_ih_d12, W_hh_d12,
           b_d12, W_a, W_out, b_out):
    B, S = x.shape
    T = y.shape[1]
    H = tgt_emb.shape[1]
    VT = W_out.shape[0]

    seq = src_emb[x.T]
    hs, cs = [], []
    seq, hf, cf = _bilstm_layer(seq, W_ih_e0, W_hh_e0, b_e0)
    hs.append(hf); cs.append(cf)
    for l in range(2):
        seq, hf, cf = _bilstm_layer(seq, W_ih_e12[l], W_hh_e12[l], b_e12[l])
        hs.append(hf); cs.append(cf)
    enc_seq = seq

    h_init = jnp.stack([jnp.concatenate([h[0], h[1]], -1) for h in hs])
    c_init = jnp.stack([jnp.concatenate([c[0], c[1]], -1) for c in cs])

    emb_y = tgt_emb[y[:, :-1].T]
    hs_top = _decoder(emb_y, enc_seq, W_a, W_ih_d0, W_hh_d0,
                      b_d0.reshape(1, -1), W_ih_d12, W_hh_d12, b_d12,
                      h_init, c_init)

    rows = hs_top.transpose(1, 0, 2).reshape(B * (T - 1), H)
    logits = _projection(rows, W_out, b_out.reshape(1, VT))
    return logits.reshape(B, T - 1, VT)
